# baseline (speedup 1.0000x reference)
import jax
import jax.numpy as jnp
from jax import lax
from jax.experimental import pallas as pl
from jax.experimental.pallas import tpu as pltpu

_BB = 32


def _writer_kernel(slot_ref, hidden_ref, wk_ref, bk_ref, wv_ref, bv_ref,
                   wg_ref, bg_ref, keys_ref, vals_ref,
                   out_keys_ref, out_vals_ref, gate_ref):
    S = keys_ref.shape[1]
    slot = slot_ref[0]

    h = hidden_ref[...]
    dn = (((1,), (1,)), ((), ()))
    nk = jnp.tanh(lax.dot_general(h, wk_ref[...], dn,
                                  preferred_element_type=jnp.float32)
                  + bk_ref[...])
    nv = jnp.tanh(lax.dot_general(h, wv_ref[...], dn,
                                  preferred_element_type=jnp.float32)
                  + bv_ref[...])
    g = jax.nn.sigmoid(lax.dot_general(h, wg_ref[...], dn,
                                       preferred_element_type=jnp.float32)
                       + bg_ref[...])

    in_k = keys_ref[...]
    in_v = vals_ref[...]
    cur_k = keys_ref[:, slot, :]
    cur_v = vals_ref[:, slot, :]
    blend_k = cur_k * (1.0 - g) + nk * g
    blend_v = cur_v * (1.0 - g) + nv * g

    sel = lax.broadcasted_iota(jnp.int32, (1, S, 1), 1) == slot
    out_keys_ref[...] = jnp.where(sel, blend_k[:, None, :], in_k)
    out_vals_ref[...] = jnp.where(sel, blend_v[:, None, :], in_v)
    gate_ref[...] = g[:, :gate_ref.shape[1]]


def kernel(hidden, working_keys, working_values, step, Wk, bk, Wv, bv, Wg, bg):
    B, S, D = working_keys.shape
    slot = (jnp.asarray(step, jnp.int32) % S).reshape(1)

    smem = pl.BlockSpec(memory_space=pltpu.MemorySpace.SMEM)
    full = lambda shape: pl.BlockSpec(shape, lambda i: (0,) * len(shape))
    bblk = pl.BlockSpec((_BB, S, D), lambda i: (i, 0, 0))

    out_keys, out_vals, gate = pl.pallas_call(
        _writer_kernel,
        grid=(B // _BB,),
        out_shape=[
            jax.ShapeDtypeStruct((B, S, D), jnp.float32),
            jax.ShapeDtypeStruct((B, S, D), jnp.float32),
            jax.ShapeDtypeStruct((B, 128), jnp.float32),
        ],
        in_specs=[
            smem,
            pl.BlockSpec((_BB, D), lambda i: (i, 0)),
            full((D, D)), full((1, D)), full((D, D)), full((1, D)),
            full((D, D)), full((1, D)),
            bblk, bblk,
        ],
        out_specs=[bblk, bblk, pl.BlockSpec((_BB, 128), lambda i: (i, 0))],
    )(slot, hidden, Wk, bk.reshape(1, D), Wv, bv.reshape(1, D),
      jnp.broadcast_to(Wg, (D, D)), jnp.broadcast_to(bg.reshape(1, 1), (1, D)),
      working_keys, working_values)

    return (out_keys, out_vals, gate[:, 0])


# speedup vs baseline: 1.0141x; 1.0141x over previous
import jax
import jax.numpy as jnp
from jax import lax
from jax.experimental import pallas as pl
from jax.experimental.pallas import tpu as pltpu

_BB = 64


def _writer_kernel(slot_ref, hidden_ref, wk_ref, bk_ref, wv_ref, bv_ref,
                   wg_ref, bg_ref, keys_ref, vals_ref,
                   out_keys_ref, out_vals_ref, gate_ref):
    S = keys_ref.shape[1]
    slot = slot_ref[0]

    h = hidden_ref[...]
    dn = (((1,), (1,)), ((), ()))
    nk = jnp.tanh(lax.dot_general(h, wk_ref[...], dn,
                                  preferred_element_type=jnp.float32)
                  + bk_ref[...])
    nv = jnp.tanh(lax.dot_general(h, wv_ref[...], dn,
                                  preferred_element_type=jnp.float32)
                  + bv_ref[...])
    g = jax.nn.sigmoid(lax.dot_general(h, wg_ref[...], dn,
                                       preferred_element_type=jnp.float32)
                       + bg_ref[...])

    in_k = keys_ref[...]
    in_v = vals_ref[...]
    cur_k = keys_ref[:, slot, :]
    cur_v = vals_ref[:, slot, :]
    blend_k = cur_k * (1.0 - g) + nk * g
    blend_v = cur_v * (1.0 - g) + nv * g

    sel = lax.broadcasted_iota(jnp.int32, (1, S, 1), 1) == slot
    out_keys_ref[...] = jnp.where(sel, blend_k[:, None, :], in_k)
    out_vals_ref[...] = jnp.where(sel, blend_v[:, None, :], in_v)
    gate_ref[...] = g[:, :gate_ref.shape[1]]


def kernel(hidden, working_keys, working_values, step, Wk, bk, Wv, bv, Wg, bg):
    B, S, D = working_keys.shape
    slot = (jnp.asarray(step, jnp.int32) % S).reshape(1)

    smem = pl.BlockSpec(memory_space=pltpu.MemorySpace.SMEM)
    full = lambda shape: pl.BlockSpec(shape, lambda i: (0,) * len(shape))
    bblk = pl.BlockSpec((_BB, S, D), lambda i: (i, 0, 0))

    out_keys, out_vals, gate = pl.pallas_call(
        _writer_kernel,
        grid=(B // _BB,),
        out_shape=[
            jax.ShapeDtypeStruct((B, S, D), jnp.float32),
            jax.ShapeDtypeStruct((B, S, D), jnp.float32),
            jax.ShapeDtypeStruct((B, 128), jnp.float32),
        ],
        in_specs=[
            smem,
            pl.BlockSpec((_BB, D), lambda i: (i, 0)),
            full((D, D)), full((1, D)), full((D, D)), full((1, D)),
            full((D, D)), full((1, D)),
            bblk, bblk,
        ],
        out_specs=[bblk, bblk, pl.BlockSpec((_BB, 128), lambda i: (i, 0))],
    )(slot, hidden, Wk, bk.reshape(1, D), Wv, bv.reshape(1, D),
      jnp.broadcast_to(Wg, (D, D)), jnp.broadcast_to(bg.reshape(1, 1), (1, D)),
      working_keys, working_values)

    return (out_keys, out_vals, gate[:, 0])
